# Initial kernel scaffold; baseline (speedup 1.0000x reference)
import functools

import jax
import jax.numpy as jnp
from jax import lax
from jax.experimental import pallas as pl
from jax.experimental.pallas import tpu as pltpu
from jax.experimental.pallas import tpu_sc as plsc

N = 10000
E = 320000
D = 128
C = 16
NC = 2
NS = 16
TILES = NC * NS
EPT = E // TILES
CHUNK = 80
NCHUNK = EPT // CHUNK
RPT = N // NS
DEGW = 16
ZROWS = 125

_sc_mesh = plsc.VectorSubcoreMesh(core_axis_name="c", subcore_axis_name="s")


@functools.partial(
    pl.kernel,
    mesh=_sc_mesh,
    out_type=(
        jax.ShapeDtypeStruct((NC * N, D), jnp.float32),
        jax.ShapeDtypeStruct((NC * N, DEGW), jnp.float32),
    ),
    scratch_types=[
        pltpu.VMEM((NCHUNK, CHUNK), jnp.int32),
        pltpu.VMEM((NCHUNK, CHUNK), jnp.int32),
        pltpu.VMEM((CHUNK, D), jnp.float32),
        pltpu.VMEM((CHUNK, DEGW), jnp.float32),
        pltpu.VMEM((ZROWS, D), jnp.float32),
        pltpu.VMEM((RPT, DEGW), jnp.float32),
        pltpu.VMEM_SHARED((N, D), jnp.float32),
        pltpu.VMEM_SHARED((N, DEGW), jnp.float32),
        pltpu.SemaphoreType.DMA,
    ],
)
def _sc_aggregate(src_hbm, dst_hbm, feat_hbm, agg_out, deg_out,
                  src_v, dst_v, rows_v, ones_v, zacc_v, zdeg_v,
                  acc_sh, deg_sh, sem):
    cid = lax.axis_index("c")
    sid = lax.axis_index("s")
    tid = cid * NS + sid

    zeros16 = jnp.zeros((16,), jnp.float32)
    ones16 = jnp.ones((16,), jnp.float32)

    def init_ones(i, carry):
        ones_v[i] = ones16
        return carry
    lax.fori_loop(0, CHUNK, init_ones, 0)

    def init_zacc(i, carry):
        for c in range(D // 16):
            zacc_v[i, pl.ds(c * 16, 16)] = zeros16
        return carry
    lax.fori_loop(0, ZROWS, init_zacc, 0)

    def init_zdeg(i, carry):
        zdeg_v[i] = zeros16
        return carry
    lax.fori_loop(0, RPT, init_zdeg, 0)

    pltpu.sync_copy(src_hbm.at[pl.ds(tid * NCHUNK, NCHUNK)], src_v)
    pltpu.sync_copy(dst_hbm.at[pl.ds(tid * NCHUNK, NCHUNK)], dst_v)

    for k in range(RPT // ZROWS):
        pltpu.sync_copy(zacc_v, acc_sh.at[pl.ds(sid * RPT + k * ZROWS, ZROWS)])
    pltpu.sync_copy(zdeg_v, deg_sh.at[pl.ds(sid * RPT, RPT)])
    plsc.subcore_barrier()

    def body(j, carry):
        pltpu.async_copy(feat_hbm.at[src_v.at[j]], rows_v, sem).wait()
        pltpu.sync_copy(rows_v, acc_sh.at[dst_v.at[j]], add=True)
        pltpu.sync_copy(ones_v, deg_sh.at[dst_v.at[j]], add=True)
        return carry
    lax.fori_loop(0, NCHUNK, body, 0)

    plsc.subcore_barrier()

    row0 = cid * N + sid * RPT
    pltpu.sync_copy(acc_sh.at[pl.ds(sid * RPT, RPT)], agg_out.at[pl.ds(row0, RPT)])
    pltpu.sync_copy(deg_sh.at[pl.ds(sid * RPT, RPT)], deg_out.at[pl.ds(row0, RPT)])


def _tc_body(f_ref, a0_ref, a1_ref, d0_ref, d1_ref, w1_ref, w2_ref, wc_ref,
             out_ref):
    f = f_ref[...]
    a = a0_ref[...] + a1_ref[...]
    d = d0_ref[...][:, 0:1] + d1_ref[...][:, 0:1]
    mean = a / jnp.maximum(d, 1.0)
    h = jnp.dot(f, w1_ref[...], preferred_element_type=jnp.float32)
    h += jnp.dot(mean, w2_ref[...], preferred_element_type=jnp.float32)
    h = jnp.maximum(h, 0.0)
    s = jnp.dot(h, wc_ref[...], preferred_element_type=jnp.float32)
    m = jnp.max(s, axis=1, keepdims=True)
    lse = jnp.log(jnp.sum(jnp.exp(s - m), axis=1, keepdims=True))
    out_ref[...] = s - m - lse


_TCB = 2000


def _tc_head(features, agg_p, deg_p, w1, w2, wc):
    nblk = N // _TCB
    return pl.pallas_call(
        _tc_body,
        grid=(nblk,),
        in_specs=[
            pl.BlockSpec((_TCB, D), lambda i: (i, 0)),
            pl.BlockSpec((_TCB, D), lambda i: (i, 0)),
            pl.BlockSpec((_TCB, D), lambda i: (i + N // _TCB, 0)),
            pl.BlockSpec((_TCB, DEGW), lambda i: (i, 0)),
            pl.BlockSpec((_TCB, DEGW), lambda i: (i + N // _TCB, 0)),
            pl.BlockSpec((D, D), lambda i: (0, 0)),
            pl.BlockSpec((D, D), lambda i: (0, 0)),
            pl.BlockSpec((D, C), lambda i: (0, 0)),
        ],
        out_specs=pl.BlockSpec((_TCB, C), lambda i: (i, 0)),
        out_shape=jax.ShapeDtypeStruct((N, C), jnp.float32),
    )(features, agg_p, agg_p, deg_p, deg_p, w1, w2, wc)


def kernel(features, adj, W_enc, weight):
    adj = adj.astype(jnp.int32)
    src = adj[0].reshape(TILES * NCHUNK, CHUNK)
    dst = adj[1].reshape(TILES * NCHUNK, CHUNK)
    agg_p, deg_p = _sc_aggregate(src, dst, features)
    return _tc_head(features, agg_p, deg_p, W_enc[:D], W_enc[D:], weight)


# speedup vs baseline: 8.3143x; 8.3143x over previous
import functools

import jax
import jax.numpy as jnp
from jax import lax
from jax.experimental import pallas as pl
from jax.experimental.pallas import tpu as pltpu
from jax.experimental.pallas import tpu_sc as plsc

N = 10000
E = 320000
D = 128
C = 16
NC = 2
NS = 16
TILES = NC * NS
EPT = E // TILES
CHUNK = 80
NCHUNK = EPT // CHUNK
NP = 10240
RPT = NP // NS
DEGW = 16
ZROWS = 128

_sc_mesh = plsc.VectorSubcoreMesh(core_axis_name="c", subcore_axis_name="s")


@functools.partial(
    pl.kernel,
    mesh=_sc_mesh,
    compiler_params=pltpu.CompilerParams(use_tc_tiling_on_sc=False),
    out_type=(
        jax.ShapeDtypeStruct((NP, D), jnp.float32),
        jax.ShapeDtypeStruct((NP, D), jnp.float32),
    ),
    scratch_types=[
        pltpu.VMEM((NCHUNK, CHUNK), jnp.int32),
        pltpu.VMEM((NCHUNK, CHUNK), jnp.int32),
        pltpu.VMEM((CHUNK, D), jnp.float32),
        pltpu.VMEM((ZROWS, D), jnp.float32),
        pltpu.VMEM_SHARED((NP, D), jnp.float32),
        pltpu.SemaphoreType.DMA,
    ],
)
def _sc_aggregate(src_hbm, dst_hbm, feat_hbm, agg0_out, agg1_out,
                  src_v, dst_v, rows_v, zacc_v, acc_sh, sem):
    cid = lax.axis_index("c")
    sid = lax.axis_index("s")
    tid = cid * NS + sid

    zeros16 = jnp.zeros((16,), jnp.float32)

    def init_zacc(i, carry):
        for c in range(D // 16):
            zacc_v[i, pl.ds(c * 16, 16)] = zeros16
        return carry
    lax.fori_loop(0, ZROWS, init_zacc, 0)

    pltpu.sync_copy(src_hbm.at[tid], src_v)
    pltpu.sync_copy(dst_hbm.at[tid], dst_v)

    zbase = pl.multiple_of(sid * RPT, 8)
    for k in range(RPT // ZROWS):
        pltpu.sync_copy(zacc_v, acc_sh.at[pl.ds(zbase + k * ZROWS, ZROWS)])
    plsc.subcore_barrier()

    def body(j, carry):
        pltpu.async_copy(feat_hbm.at[src_v.at[j]], rows_v, sem).wait()
        pltpu.sync_copy(rows_v, acc_sh.at[dst_v.at[j]], add=True)
        return carry
    lax.fori_loop(0, NCHUNK, body, 0)

    plsc.subcore_barrier()

    @pl.when(cid == 0)
    def _():
        pltpu.sync_copy(acc_sh.at[pl.ds(zbase, RPT)], agg0_out.at[pl.ds(zbase, RPT)])

    @pl.when(cid == 1)
    def _():
        pltpu.sync_copy(acc_sh.at[pl.ds(zbase, RPT)], agg1_out.at[pl.ds(zbase, RPT)])


@functools.partial(
    pl.kernel,
    mesh=_sc_mesh,
    compiler_params=pltpu.CompilerParams(use_tc_tiling_on_sc=False),
    out_type=(
        jax.ShapeDtypeStruct((NP, DEGW), jnp.float32),
        jax.ShapeDtypeStruct((NP, DEGW), jnp.float32),
    ),
    scratch_types=[
        pltpu.VMEM((NCHUNK, CHUNK), jnp.int32),
        pltpu.VMEM((CHUNK, DEGW), jnp.float32),
        pltpu.VMEM((RPT, DEGW), jnp.float32),
        pltpu.VMEM_SHARED((NP, DEGW), jnp.float32),
    ],
)
def _sc_degree(dst_hbm, deg0_out, deg1_out, dst_v, ones_v, zdeg_v, deg_sh):
    cid = lax.axis_index("c")
    sid = lax.axis_index("s")
    tid = cid * NS + sid

    zeros16 = jnp.zeros((16,), jnp.float32)
    ones16 = jnp.ones((16,), jnp.float32)

    def init_ones(i, carry):
        ones_v[i] = ones16
        return carry
    lax.fori_loop(0, CHUNK, init_ones, 0)

    def init_zdeg(i, carry):
        zdeg_v[i] = zeros16
        return carry
    lax.fori_loop(0, RPT, init_zdeg, 0)

    pltpu.sync_copy(dst_hbm.at[tid], dst_v)

    zbase = pl.multiple_of(sid * RPT, 8)
    pltpu.sync_copy(zdeg_v, deg_sh.at[pl.ds(zbase, RPT)])
    plsc.subcore_barrier()

    def body(j, carry):
        pltpu.sync_copy(ones_v, deg_sh.at[dst_v.at[j]], add=True)
        return carry
    lax.fori_loop(0, NCHUNK, body, 0)

    plsc.subcore_barrier()

    @pl.when(cid == 0)
    def _():
        pltpu.sync_copy(deg_sh.at[pl.ds(zbase, RPT)], deg0_out.at[pl.ds(zbase, RPT)])

    @pl.when(cid == 1)
    def _():
        pltpu.sync_copy(deg_sh.at[pl.ds(zbase, RPT)], deg1_out.at[pl.ds(zbase, RPT)])


def _tc_body(f_ref, a0_ref, a1_ref, d0_ref, d1_ref, w1_ref, w2_ref, wc_ref,
             out_ref):
    f = f_ref[...]
    a = a0_ref[...] + a1_ref[...]
    d = d0_ref[...][:, 0:1] + d1_ref[...][:, 0:1]
    mean = a / jnp.maximum(d, 1.0)
    h = jnp.dot(f, w1_ref[...], preferred_element_type=jnp.float32)
    h += jnp.dot(mean, w2_ref[...], preferred_element_type=jnp.float32)
    h = jnp.maximum(h, 0.0)
    s = jnp.dot(h, wc_ref[...], preferred_element_type=jnp.float32)
    m = jnp.max(s, axis=1, keepdims=True)
    lse = jnp.log(jnp.sum(jnp.exp(s - m), axis=1, keepdims=True))
    out_ref[...] = s - m - lse


_TCB = 2000


def _tc_head(features, agg0, agg1, deg0, deg1, w1, w2, wc):
    nblk = N // _TCB
    return pl.pallas_call(
        _tc_body,
        grid=(nblk,),
        in_specs=[
            pl.BlockSpec((_TCB, D), lambda i: (i, 0)),
            pl.BlockSpec((_TCB, D), lambda i: (i, 0)),
            pl.BlockSpec((_TCB, D), lambda i: (i, 0)),
            pl.BlockSpec((_TCB, DEGW), lambda i: (i, 0)),
            pl.BlockSpec((_TCB, DEGW), lambda i: (i, 0)),
            pl.BlockSpec((D, D), lambda i: (0, 0)),
            pl.BlockSpec((D, D), lambda i: (0, 0)),
            pl.BlockSpec((D, C), lambda i: (0, 0)),
        ],
        out_specs=pl.BlockSpec((_TCB, C), lambda i: (i, 0)),
        out_shape=jax.ShapeDtypeStruct((N, C), jnp.float32),
    )(features, agg0, agg1, deg0, deg1, w1, w2, wc)


def kernel(features, adj, W_enc, weight):
    adj = adj.astype(jnp.int32)
    src = adj[0].reshape(TILES, NCHUNK, CHUNK)
    dst = adj[1].reshape(TILES, NCHUNK, CHUNK)
    deg0, deg1 = _sc_degree(dst)
    agg0, agg1 = _sc_aggregate(src, dst, features)
    return _tc_head(features, agg0, agg1, deg0, deg1, W_enc[:D], W_enc[D:], weight)


# speedup vs baseline: 12.8363x; 1.5439x over previous
import functools

import jax
import jax.numpy as jnp
from jax import lax
from jax.experimental import pallas as pl
from jax.experimental.pallas import tpu as pltpu
from jax.experimental.pallas import tpu_sc as plsc

N = 10000
E = 320000
D = 128
C = 16
NC = 2
NS = 16
TILES = NC * NS
EPT = E // TILES
CHUNK = 80
NCHUNK = EPT // CHUNK
NP = 10240
RPT = NP // NS
DEGW = 16
ZROWS = 128

_sc_mesh = plsc.VectorSubcoreMesh(core_axis_name="c", subcore_axis_name="s")


@functools.partial(
    pl.kernel,
    mesh=_sc_mesh,
    compiler_params=pltpu.CompilerParams(use_tc_tiling_on_sc=False),
    out_type=(
        jax.ShapeDtypeStruct((NP, D), jnp.float32),
        jax.ShapeDtypeStruct((NP, D), jnp.float32),
        jax.ShapeDtypeStruct((NP, DEGW), jnp.float32),
        jax.ShapeDtypeStruct((NP, DEGW), jnp.float32),
    ),
    scratch_types=[
        pltpu.VMEM((NCHUNK, CHUNK), jnp.int32),
        pltpu.VMEM((2, CHUNK), jnp.int32),
        pltpu.VMEM((2, CHUNK), jnp.int32),
        pltpu.VMEM((CHUNK, D), jnp.float32),
        pltpu.VMEM((CHUNK, D), jnp.float32),
        pltpu.VMEM((CHUNK, DEGW), jnp.float32),
        pltpu.VMEM((CHUNK, DEGW), jnp.float32),
        pltpu.VMEM_SHARED((NP, D), jnp.float32),
        pltpu.VMEM_SHARED((NP, DEGW), jnp.float32),
        pltpu.SemaphoreType.DMA,
        pltpu.SemaphoreType.DMA,
    ],
)
def _sc_aggregate(pk_hbm, feat_hbm, agg0_out, agg1_out, deg0_out, deg1_out,
                  pk_v, src_r, dst_r, rows_a, rows_b, ones_v, zdeg_v,
                  acc_sh, deg_sh, sem_a, sem_b):
    cid = lax.axis_index("c")
    sid = lax.axis_index("s")
    tid = cid * NS + sid

    zeros16 = jnp.zeros((16,), jnp.float32)
    ones16 = jnp.ones((16,), jnp.float32)

    pltpu.sync_copy(pk_hbm.at[tid], pk_v)

    def init_small(i, carry):
        ones_v[i] = ones16
        zdeg_v[i] = zeros16
        return carry
    lax.fori_loop(0, CHUNK, init_small, 0)

    def init_za(i, carry):
        for c in range(D // 16):
            rows_a[i, pl.ds(c * 16, 16)] = zeros16
        return carry
    lax.fori_loop(0, CHUNK, init_za, 0)

    zbase = pl.multiple_of(sid * RPT, 8)
    for k in range(RPT // CHUNK):
        pltpu.sync_copy(rows_a, acc_sh.at[pl.ds(zbase + k * CHUNK, CHUNK)])
    for k in range(RPT // CHUNK):
        pltpu.sync_copy(zdeg_v, deg_sh.at[pl.ds(zbase + k * CHUNK, CHUNK)])
    plsc.subcore_barrier()

    def unpack(j, s):
        for g in range(CHUNK // 16):
            w = pk_v[j, pl.ds(g * 16, 16)]
            src_r[s, pl.ds(g * 16, 16)] = w & 0xFFFF
            dst_r[s, pl.ds(g * 16, 16)] = lax.shift_right_logical(w, 16)

    unpack(0, 0)
    unpack(1, 1)
    pltpu.async_copy(feat_hbm.at[src_r.at[0]], rows_a, sem_a)
    pltpu.async_copy(feat_hbm.at[src_r.at[1]], rows_b, sem_b)

    def body(i, carry):
        j = 2 * i
        pltpu.make_async_copy(feat_hbm.at[src_r.at[0]], rows_a, sem_a).wait()
        pltpu.sync_copy(rows_a, acc_sh.at[dst_r.at[0]], add=True)
        pltpu.sync_copy(ones_v, deg_sh.at[dst_r.at[0]], add=True)

        @pl.when(j + 2 < NCHUNK)
        def _():
            unpack(j + 2, 0)
            pltpu.async_copy(feat_hbm.at[src_r.at[0]], rows_a, sem_a)

        pltpu.make_async_copy(feat_hbm.at[src_r.at[1]], rows_b, sem_b).wait()
        pltpu.sync_copy(rows_b, acc_sh.at[dst_r.at[1]], add=True)
        pltpu.sync_copy(ones_v, deg_sh.at[dst_r.at[1]], add=True)

        @pl.when(j + 3 < NCHUNK)
        def _():
            unpack(j + 3, 1)
            pltpu.async_copy(feat_hbm.at[src_r.at[1]], rows_b, sem_b)

        return carry
    lax.fori_loop(0, (NCHUNK - 1) // 2, body, 0)

    pltpu.make_async_copy(feat_hbm.at[src_r.at[0]], rows_a, sem_a).wait()
    pltpu.sync_copy(rows_a, acc_sh.at[dst_r.at[0]], add=True)
    pltpu.sync_copy(ones_v, deg_sh.at[dst_r.at[0]], add=True)

    plsc.subcore_barrier()

    @pl.when(cid == 0)
    def _():
        pltpu.sync_copy(acc_sh.at[pl.ds(zbase, RPT)], agg0_out.at[pl.ds(zbase, RPT)])
        pltpu.sync_copy(deg_sh.at[pl.ds(zbase, RPT)], deg0_out.at[pl.ds(zbase, RPT)])

    @pl.when(cid == 1)
    def _():
        pltpu.sync_copy(acc_sh.at[pl.ds(zbase, RPT)], agg1_out.at[pl.ds(zbase, RPT)])
        pltpu.sync_copy(deg_sh.at[pl.ds(zbase, RPT)], deg1_out.at[pl.ds(zbase, RPT)])


def _tc_body(f_ref, a0_ref, a1_ref, d0_ref, d1_ref, w1_ref, w2_ref, wc_ref,
             out_ref):
    f = f_ref[...]
    a = a0_ref[...] + a1_ref[...]
    d = d0_ref[...][:, 0:1] + d1_ref[...][:, 0:1]
    mean = a / jnp.maximum(d, 1.0)
    h = jnp.dot(f, w1_ref[...], preferred_element_type=jnp.float32)
    h += jnp.dot(mean, w2_ref[...], preferred_element_type=jnp.float32)
    h = jnp.maximum(h, 0.0)
    s = jnp.dot(h, wc_ref[...], preferred_element_type=jnp.float32)
    m = jnp.max(s, axis=1, keepdims=True)
    lse = jnp.log(jnp.sum(jnp.exp(s - m), axis=1, keepdims=True))
    out_ref[...] = s - m - lse


_TCB = 2000


def _tc_head(features, agg0, agg1, deg0, deg1, w1, w2, wc):
    nblk = N // _TCB
    return pl.pallas_call(
        _tc_body,
        grid=(nblk,),
        in_specs=[
            pl.BlockSpec((_TCB, D), lambda i: (i, 0)),
            pl.BlockSpec((_TCB, D), lambda i: (i, 0)),
            pl.BlockSpec((_TCB, D), lambda i: (i, 0)),
            pl.BlockSpec((_TCB, DEGW), lambda i: (i, 0)),
            pl.BlockSpec((_TCB, DEGW), lambda i: (i, 0)),
            pl.BlockSpec((D, D), lambda i: (0, 0)),
            pl.BlockSpec((D, D), lambda i: (0, 0)),
            pl.BlockSpec((D, C), lambda i: (0, 0)),
        ],
        out_specs=pl.BlockSpec((_TCB, C), lambda i: (i, 0)),
        out_shape=jax.ShapeDtypeStruct((N, C), jnp.float32),
    )(features, agg0, agg1, deg0, deg1, w1, w2, wc)


def kernel(features, adj, W_enc, weight):
    adj = adj.astype(jnp.int32)
    packed = jnp.bitwise_or(jnp.left_shift(adj[1], 16), adj[0])
    packed = packed.reshape(TILES, NCHUNK, CHUNK)
    agg0, agg1, deg0, deg1 = _sc_aggregate(packed, features)
    return _tc_head(features, agg0, agg1, deg0, deg1, W_enc[:D], W_enc[D:], weight)
